# Initial kernel scaffold; baseline (speedup 1.0000x reference)
import jax
import jax.numpy as jnp
from jax.experimental import pallas as pl


def kernel(x, durations, val_ind):
    raise NotImplementedError("write your pallas kernel here")


# speedup vs baseline: 2.0898x; 2.0898x over previous
import functools

import jax
import jax.numpy as jnp
from jax import lax
from jax.experimental import pallas as pl
from jax.experimental.pallas import tpu as pltpu
from jax.experimental.pallas import tpu_sc as plsc


@functools.cache
def _build(B, P, F, D):
    info = plsc.get_sparse_core_info()
    NC, NS, L = info.num_cores, info.num_subcores, info.num_lanes
    NW = NC * NS
    rows_w = (B * F) // NW
    CHUNK = 64
    nchunk = rows_w // CHUNK
    mesh = plsc.VectorSubcoreMesh(core_axis_name="c", subcore_axis_name="s")

    @functools.partial(
        pl.kernel,
        mesh=mesh,
        out_type=[
            jax.ShapeDtypeStruct((B * F, D), jnp.float32),
            jax.ShapeDtypeStruct((B * F,), jnp.int32),
        ],
        scratch_types=[
            pltpu.VMEM((rows_w,), jnp.int32),
            pltpu.VMEM((rows_w,), jnp.int32),
            pltpu.VMEM((2, CHUNK, D), jnp.float32),
            pltpu.SemaphoreType.DMA,
            pltpu.SemaphoreType.DMA,
        ],
    )
    def lr(x_hbm, vi_hbm, out_hbm, msk_hbm, idx_v, msk_v, rows_v, gsem, ssem):
        wid = lax.axis_index("s") * NC + lax.axis_index("c")
        base = wid * rows_w
        pltpu.sync_copy(vi_hbm.at[pl.ds(base, rows_w)], idx_v)

        def body(i, carry):
            v = idx_v[pl.ds(i * L, L)]
            msk_v[pl.ds(i * L, L)] = jnp.where(
                v != P - 1,
                jnp.full((L,), 1, jnp.int32),
                jnp.full((L,), 0, jnp.int32),
            )
            boff = ((base + i * L) // F) * P
            idx_v[pl.ds(i * L, L)] = v + boff
            return carry

        lax.fori_loop(0, rows_w // L, body, 0)
        pltpu.sync_copy(msk_v, msk_hbm.at[pl.ds(base, rows_w)])

        scatters = []
        for g in range(nchunk):
            buf = g & 1
            if g >= 2:
                scatters[g - 2].wait()
            pltpu.async_copy(
                x_hbm.at[idx_v.at[pl.ds(g * CHUNK, CHUNK)]], rows_v.at[buf], gsem
            ).wait()
            scatters.append(pltpu.async_copy(
                rows_v.at[buf], out_hbm.at[pl.ds(base + g * CHUNK, CHUNK)], ssem))
        scatters[-2].wait()
        scatters[-1].wait()

    return lr


def kernel(x, durations, val_ind):
    del durations
    B, P, D = x.shape
    F = val_ind.shape[1]
    lr = _build(B, P, F, D)
    out_flat, msk = lr(x.reshape(B * P, D), val_ind.reshape(B * F))
    return out_flat.reshape(B, F, D), msk.astype(bool).reshape(B, F, 1)


# speedup vs baseline: 2.3391x; 1.1192x over previous
import functools

import jax
import jax.numpy as jnp
from jax import lax
from jax.experimental import pallas as pl
from jax.experimental.pallas import tpu as pltpu
from jax.experimental.pallas import tpu_sc as plsc


@functools.cache
def _build(B, P, F, D):
    info = plsc.get_sparse_core_info()
    NC, NS, L = info.num_cores, info.num_subcores, info.num_lanes
    NW = NC * NS
    rows_w = (B * F) // NW
    CHUNK = 64
    NBUF = 3
    nchunk = rows_w // CHUNK
    mesh = plsc.VectorSubcoreMesh(core_axis_name="c", subcore_axis_name="s")

    @functools.partial(
        pl.kernel,
        mesh=mesh,
        out_type=[
            jax.ShapeDtypeStruct((B * F, D), jnp.float32),
            jax.ShapeDtypeStruct((B * F,), jnp.int32),
        ],
        scratch_types=[
            pltpu.VMEM((rows_w,), jnp.int32),
            pltpu.VMEM((rows_w,), jnp.int32),
            pltpu.VMEM((NBUF, CHUNK, D), jnp.float32),
            pltpu.SemaphoreType.DMA,
            pltpu.SemaphoreType.DMA,
        ],
    )
    def lr(x_hbm, vi_hbm, out_hbm, msk_hbm, idx_v, msk_v, rows_v, gsem, ssem):
        wid = lax.axis_index("s") * NC + lax.axis_index("c")
        base = wid * rows_w
        pltpu.sync_copy(vi_hbm.at[pl.ds(base, rows_w)], idx_v)

        def body(i, carry):
            v = idx_v[pl.ds(i * L, L)]
            msk_v[pl.ds(i * L, L)] = jnp.where(
                v != P - 1,
                jnp.full((L,), 1, jnp.int32),
                jnp.full((L,), 0, jnp.int32),
            )
            boff = ((base + i * L) // F) * P
            idx_v[pl.ds(i * L, L)] = v + boff
            return carry

        lax.fori_loop(0, rows_w // L, body, 0)
        pltpu.sync_copy(msk_v, msk_hbm.at[pl.ds(base, rows_w)])

        gathers, scatters = [], []

        def start_gather(g):
            gathers.append(pltpu.async_copy(
                x_hbm.at[idx_v.at[pl.ds(g * CHUNK, CHUNK)]],
                rows_v.at[g % NBUF], gsem))

        def start_scatter(g):
            gathers[g].wait()
            scatters.append(pltpu.async_copy(
                rows_v.at[g % NBUF],
                out_hbm.at[pl.ds(base + g * CHUNK, CHUNK)], ssem))

        for g in range(nchunk):
            if g >= NBUF:
                scatters[g - NBUF].wait()
            start_gather(g)
            if g >= 1:
                start_scatter(g - 1)
        start_scatter(nchunk - 1)
        for g in range(max(0, nchunk - NBUF), nchunk):
            scatters[g].wait()

    return lr


def kernel(x, durations, val_ind):
    del durations
    B, P, D = x.shape
    F = val_ind.shape[1]
    lr = _build(B, P, F, D)
    out_flat, msk = lr(x.reshape(B * P, D), val_ind.reshape(B * F))
    return out_flat.reshape(B, F, D), msk.astype(bool).reshape(B, F, 1)


# speedup vs baseline: 2.3463x; 1.0031x over previous
import functools

import jax
import jax.numpy as jnp
from jax import lax
from jax.experimental import pallas as pl
from jax.experimental.pallas import tpu as pltpu
from jax.experimental.pallas import tpu_sc as plsc


@functools.cache
def _build(B, P, F, D):
    info = plsc.get_sparse_core_info()
    NC, NS, L = info.num_cores, info.num_subcores, info.num_lanes
    NW = NC * NS
    rows_w = (B * F) // NW
    CHUNK = 64
    NBUF = 3
    nchunk = rows_w // CHUNK
    mesh = plsc.VectorSubcoreMesh(core_axis_name="c", subcore_axis_name="s")

    @functools.partial(
        pl.kernel,
        mesh=mesh,
        out_type=[
            jax.ShapeDtypeStruct((B * F, D), jnp.float32),
            jax.ShapeDtypeStruct((B * F,), jnp.int32),
        ],
        scratch_types=[
            pltpu.VMEM((rows_w,), jnp.int32),
            pltpu.VMEM((rows_w,), jnp.int32),
            pltpu.VMEM((NBUF, CHUNK, D), jnp.float32),
            pltpu.SemaphoreType.DMA,
            pltpu.SemaphoreType.DMA,
        ],
    )
    def lr(x_hbm, vi_hbm, out_hbm, msk_hbm, idx_v, msk_v, rows_v, gsem, ssem):
        wid = lax.axis_index("s") * NC + lax.axis_index("c")
        base = wid * rows_w
        pltpu.sync_copy(vi_hbm.at[pl.ds(base, rows_w)], idx_v)

        def build(g):
            for j in range(CHUNK // L):
                i = g * (CHUNK // L) + j
                v = idx_v[pl.ds(i * L, L)]
                msk_v[pl.ds(i * L, L)] = jnp.where(
                    v != P - 1,
                    jnp.full((L,), 1, jnp.int32),
                    jnp.full((L,), 0, jnp.int32),
                )
                boff = ((base + i * L) // F) * P
                idx_v[pl.ds(i * L, L)] = v + boff

        gathers, scatters = [], []

        def start_gather(g):
            gathers.append(pltpu.async_copy(
                x_hbm.at[idx_v.at[pl.ds(g * CHUNK, CHUNK)]],
                rows_v.at[g % NBUF], gsem))

        def start_scatter(g):
            gathers[g].wait()
            scatters.append(pltpu.async_copy(
                rows_v.at[g % NBUF],
                out_hbm.at[pl.ds(base + g * CHUNK, CHUNK)], ssem))

        for g in range(nchunk):
            build(g)
            if g >= NBUF:
                scatters[g - NBUF].wait()
            start_gather(g)
            if g >= 1:
                start_scatter(g - 1)
        msk_copy = pltpu.async_copy(msk_v, msk_hbm.at[pl.ds(base, rows_w)], gsem)
        start_scatter(nchunk - 1)
        for g in range(max(0, nchunk - NBUF), nchunk):
            scatters[g].wait()
        msk_copy.wait()

    return lr


def kernel(x, durations, val_ind):
    del durations
    B, P, D = x.shape
    F = val_ind.shape[1]
    lr = _build(B, P, F, D)
    out_flat, msk = lr(x.reshape(B * P, D), val_ind.reshape(B * F))
    return out_flat.reshape(B, F, D), msk.astype(bool).reshape(B, F, 1)
